# baseline (speedup 1.0000x reference)
import functools

import jax
import jax.numpy as jnp
from jax.experimental import pallas as pl
from jax.experimental.pallas import tpu as pltpu

N = 10000
E = 320000
HEADS = 8
HID = 64
HD = HEADS * HID


def _bn(x, g, b):
    m = jnp.mean(x, axis=0)
    v = jnp.var(x, axis=0)
    return g * (x - m) / jnp.sqrt(v + 1e-5) + b


def _gat(x, src, dst, Wl, bl, Wr, br, att, bias):
    n = x.shape[0]
    xl = (x @ Wl + bl).reshape(n, HEADS, HID)
    xr = (x @ Wr + br).reshape(n, HEADS, HID)
    xj = xl[src]
    xi = xr[dst]
    e = jax.nn.leaky_relu(xi + xj, 0.2)
    logits = jnp.einsum('ehc,hc->eh', e, att)
    mx = jax.ops.segment_max(logits, dst, num_segments=n)
    ex = jnp.exp(logits - mx[dst])
    den = jax.ops.segment_sum(ex, dst, num_segments=n)
    alpha = ex / den[dst]
    out = jax.ops.segment_sum(alpha[:, :, None] * xj, dst, num_segments=n)
    return out.reshape(n, HEADS * HID) + bias


def _head_kernel(h_ref, p1w_ref, p1g_ref, p1b_ref, p2w_ref, p2g_ref, p2b_ref,
                 p3w_ref, p3b_ref, out_ref):
    h = h_ref[...]
    y = h @ p1w_ref[...]
    m = jnp.mean(y, axis=0)
    v = jnp.mean((y - m[None, :]) ** 2, axis=0)
    y = p1g_ref[...] * (y - m[None, :]) / jnp.sqrt(v[None, :] + 1e-5) + p1b_ref[...]
    y = jnp.maximum(y, 0.0)
    z = y @ p2w_ref[...]
    m2 = jnp.mean(z, axis=0)
    v2 = jnp.mean((z - m2[None, :]) ** 2, axis=0)
    z = p2g_ref[...] * (z - m2[None, :]) / jnp.sqrt(v2[None, :] + 1e-5) + p2b_ref[...]
    z = jnp.maximum(z, 0.0)
    out_ref[...] = z @ p3w_ref[...] + p3b_ref[...]


def _mlp_head(h, p):
    out = pl.pallas_call(
        _head_kernel,
        out_shape=jax.ShapeDtypeStruct((N, 1), jnp.float32),
    )(h, p['p1_W'], p['p1_g'], p['p1_beta'], p['p2_W'], p['p2_g'],
      p['p2_beta'], p['p3_W'], p['p3_b'])
    return out[:, 0]


def kernel(x, edge_index, edge_attr, params):
    p = params
    src, dst = edge_index[0], edge_index[1]
    loops = jnp.arange(N)
    src2 = jnp.concatenate([src, loops])
    dst2 = jnp.concatenate([dst, loops])
    a = jax.nn.relu(_bn(x @ p['ap_W'] + p['ap_b'], p['ap_g'], p['ap_beta']))
    e = jax.nn.relu(_bn(edge_attr @ p['ep_W'] + p['ep_b'], p['ep_g'], p['ep_beta']))
    cnt = jax.ops.segment_sum(jnp.ones((E,), jnp.float32), dst, num_segments=N)
    agg = jax.ops.segment_sum(e, dst, num_segments=N) / jnp.clip(cnt, 1.0)[:, None]
    msg = jax.nn.relu(_bn((a + agg) @ p['am_W'] + p['am_b'], p['am_g'], p['am_beta']))
    h = jnp.concatenate([msg, agg], axis=1)
    h = _gat(h, src2, dst2, p['g1_Wl'], p['g1_bl'], p['g1_Wr'], p['g1_br'], p['g1_att'], p['g1_bias'])
    h = jax.nn.relu(_bn(h, p['bn1_g'], p['bn1_b']))
    h = _gat(h, src2, dst2, p['g2_Wl'], p['g2_bl'], p['g2_Wr'], p['g2_br'], p['g2_att'], p['g2_bias'])
    h = jax.nn.relu(_bn(h, p['bn2_g'], p['bn2_b']))
    return _mlp_head(h, p)


# speedup vs baseline: 6.7419x; 6.7419x over previous
import functools

import jax
import jax.numpy as jnp
from jax import lax
from jax.experimental import pallas as pl
from jax.experimental.pallas import tpu as pltpu
from jax.experimental.pallas import tpu_sc as plsc

N = 10000
E = 320000
HEADS = 8
HID = 64
HD = HEADS * HID

NTILES = 32
CHUNK = 64
BLK = 2048
SRCCAP = 32768
ETOT = E + N
EPAD = ((ETOT + CHUNK + 2 * BLK + BLK - 1) // BLK) * BLK
OFFPAD = ((N + 1 + 16 + 7) // 8) * 8
TSPAD = 48


def _sload(ref, i):
    base = (i // 8) * 8
    v = ref[pl.ds(base, 16)]
    idx = jnp.full((16, 1), i - base, jnp.int32)
    dn = lax.GatherDimensionNumbers(
        offset_dims=(), collapsed_slice_dims=(0,), start_index_map=(0,))
    return lax.gather(v, idx, dn, (1,),
                      mode=lax.GatherScatterMode.PROMISE_IN_BOUNDS)[0]


def _bn(x, g, b):
    m = jnp.mean(x, axis=0)
    v = jnp.var(x, axis=0)
    return g * (x - m) / jnp.sqrt(v + 1e-5) + b


def _bcast(v, lane):
    idx = jnp.full((16, 1), lane, jnp.int32)
    dn = lax.GatherDimensionNumbers(
        offset_dims=(), collapsed_slice_dims=(0,), start_index_map=(0,))
    return lax.gather(v, idx, dn, (1,),
                      mode=lax.GatherScatterMode.PROMISE_IN_BOUNDS)


def _edge_prep(src, dst):
    loops = jnp.arange(N, dtype=jnp.int32)
    src2 = jnp.concatenate([src.astype(jnp.int32), loops])
    dst2 = jnp.concatenate([dst.astype(jnp.int32), loops])
    ids = jnp.arange(ETOT, dtype=jnp.int32)
    sd, ss, perm = lax.sort((dst2, src2, ids), num_keys=1)
    off = jnp.searchsorted(sd, jnp.arange(N + 1, dtype=jnp.int32),
                           side='left').astype(jnp.int32)
    off_pad = jnp.concatenate(
        [off, jnp.full((OFFPAD - (N + 1),), ETOT, jnp.int32)])
    src_pad = jnp.concatenate([ss, jnp.zeros((EPAD - ETOT,), jnp.int32)])
    targets = (jnp.arange(NTILES + 1, dtype=jnp.int32) * ETOT) // NTILES
    ts = jnp.searchsorted(off, targets, side='left').astype(jnp.int32)
    ts_pad = jnp.concatenate([ts, jnp.zeros((TSPAD - (NTILES + 1),), jnp.int32)])
    return src_pad, off_pad, ts_pad, perm


def _gat_body(xl_hbm, xr_hbm, att_hbm, src_hbm, off_hbm, ts_hbm, out_hbm,
              srcbuf, offbuf, tsbuf, attbuf, rowbuf, xibuf, outbuf,
              accbuf, denbuf, sem):
    wid = lax.axis_index("s") * 2 + lax.axis_index("c")
    pltpu.sync_copy(off_hbm, offbuf)
    pltpu.sync_copy(ts_hbm, tsbuf)
    pltpu.sync_copy(att_hbm, attbuf)
    n_lo = _sload(tsbuf, wid)
    n_hi = _sload(tsbuf, wid + 1)

    def _finalize(i):
        den = denbuf[...]
        for h in range(HEADS):
            db = _bcast(den, h)
            for kk in range(HID // 16):
                c0 = h * HID + kk * 16
                outbuf[0, pl.ds(c0, 16)] = accbuf[pl.ds(c0, 16)] / db
        pltpu.sync_copy(outbuf, out_hbm.at[pl.ds(i, 1)])

    def _reset():
        denbuf[...] = jnp.zeros((16,), jnp.float32)
        for kk in range(HD // 16):
            accbuf[pl.ds(kk * 16, 16)] = jnp.zeros((16,), jnp.float32)

    @pl.when(n_lo < n_hi)
    def _():
        e_lo = _sload(offbuf, n_lo)
        e_hi = _sload(offbuf, n_hi)
        alo = (e_lo // 8) * 8
        nblk = jnp.minimum((e_hi - alo + CHUNK + BLK - 1) // BLK,
                           SRCCAP // BLK)

        def stage(b, c):
            pltpu.sync_copy(src_hbm.at[pl.ds(alo + b * BLK, BLK)],
                            srcbuf.at[pl.ds(b * BLK, BLK)])
            return c
        lax.fori_loop(0, nblk, stage, 0)

        _reset()
        pltpu.sync_copy(xr_hbm.at[pl.ds(n_lo, 1)], xibuf)
        nchunk = (e_hi - alo + CHUNK - 1) // CHUNK
        lane = lax.iota(jnp.int32, 16)

        def chunk_body(k, cur):
            qbase = alo + k * CHUNK
            pltpu.async_copy(xl_hbm.at[srcbuf.at[pl.ds(k * CHUNK, CHUNK)]],
                             rowbuf, sem).wait()
            ebeg = jnp.maximum(e_lo - qbase, 0)
            ne = jnp.minimum(CHUNK, e_hi - qbase)

            def edge_body(e, cur):
                q = qbase + e
                adv = q == _sload(offbuf, cur + 1)

                @pl.when(adv)
                def _fin():
                    _finalize(cur)
                    _reset()
                    pltpu.sync_copy(xr_hbm.at[pl.ds(cur + 1, 1)], xibuf)

                cur = jnp.where(adv, cur + 1, cur)
                lvec = jnp.zeros((16,), jnp.float32)
                for h in range(HEADS):
                    sv = jnp.zeros((16,), jnp.float32)
                    for kk in range(HID // 16):
                        c0 = h * HID + kk * 16
                        xj = rowbuf[e, pl.ds(c0, 16)]
                        xi = xibuf[0, pl.ds(c0, 16)]
                        t = xi + xj
                        lk = jnp.maximum(t, 0.2 * t)
                        sv = sv + lk * attbuf[pl.ds(c0, 16)]
                    lvec = jnp.where(lane == h, jnp.sum(sv), lvec)
                ex = jnp.exp(lvec)
                denbuf[...] = denbuf[...] + ex
                for h in range(HEADS):
                    eb = _bcast(ex, h)
                    for kk in range(HID // 16):
                        c0 = h * HID + kk * 16
                        xj = rowbuf[e, pl.ds(c0, 16)]
                        accbuf[pl.ds(c0, 16)] = accbuf[pl.ds(c0, 16)] + xj * eb
                return cur

            return lax.fori_loop(ebeg, ne, edge_body, cur)

        lax.fori_loop(0, nchunk, chunk_body, n_lo)
        _finalize(n_hi - 1)


@jax.jit
def _gat_sc_call(xl, xr, attf, src_pad, off_pad, ts_pad):
    mesh = plsc.VectorSubcoreMesh(core_axis_name="c", subcore_axis_name="s")
    return pl.kernel(
        _gat_body,
        mesh=mesh,
        out_type=jax.ShapeDtypeStruct((N, HD), jnp.float32),
        compiler_params=pltpu.CompilerParams(needs_layout_passes=False),
        scratch_types=[
            pltpu.VMEM((SRCCAP,), jnp.int32),
            pltpu.VMEM((OFFPAD,), jnp.int32),
            pltpu.VMEM((TSPAD,), jnp.int32),
            pltpu.VMEM((HD,), jnp.float32),
            pltpu.VMEM((CHUNK, HD), jnp.float32),
            pltpu.VMEM((1, HD), jnp.float32),
            pltpu.VMEM((1, HD), jnp.float32),
            pltpu.VMEM((HD,), jnp.float32),
            pltpu.VMEM((16,), jnp.float32),
            pltpu.SemaphoreType.DMA,
        ],
    )(xl, xr, attf, src_pad, off_pad, ts_pad)


def _gat(x, Wl, bl, Wr, br, att, bias, src_pad, off_pad, ts_pad):
    xl = x @ Wl + bl
    xr = x @ Wr + br
    out = _gat_sc_call(xl, xr, att.reshape(HD), src_pad, off_pad, ts_pad)
    return out + bias


def _head_kernel(h_ref, p1w_ref, p1g_ref, p1b_ref, p2w_ref, p2g_ref, p2b_ref,
                 p3w_ref, p3b_ref, out_ref):
    h = h_ref[...]
    y = h @ p1w_ref[...]
    m = jnp.mean(y, axis=0)
    v = jnp.mean((y - m[None, :]) ** 2, axis=0)
    y = p1g_ref[...] * (y - m[None, :]) / jnp.sqrt(v[None, :] + 1e-5) + p1b_ref[...]
    y = jnp.maximum(y, 0.0)
    z = y @ p2w_ref[...]
    m2 = jnp.mean(z, axis=0)
    v2 = jnp.mean((z - m2[None, :]) ** 2, axis=0)
    z = p2g_ref[...] * (z - m2[None, :]) / jnp.sqrt(v2[None, :] + 1e-5) + p2b_ref[...]
    z = jnp.maximum(z, 0.0)
    out_ref[...] = z @ p3w_ref[...] + p3b_ref[...]


def _mlp_head(h, p):
    out = pl.pallas_call(
        _head_kernel,
        out_shape=jax.ShapeDtypeStruct((N, 1), jnp.float32),
    )(h, p['p1_W'], p['p1_g'], p['p1_beta'], p['p2_W'], p['p2_g'],
      p['p2_beta'], p['p3_W'], p['p3_b'])
    return out[:, 0]


def kernel(x, edge_index, edge_attr, params):
    p = params
    src, dst = edge_index[0], edge_index[1]
    src_pad, off_pad, ts_pad, _perm = _edge_prep(src, dst)
    a = jax.nn.relu(_bn(x @ p['ap_W'] + p['ap_b'], p['ap_g'], p['ap_beta']))
    e = jax.nn.relu(_bn(edge_attr @ p['ep_W'] + p['ep_b'], p['ep_g'], p['ep_beta']))
    cnt = jax.ops.segment_sum(jnp.ones((E,), jnp.float32), dst, num_segments=N)
    agg = jax.ops.segment_sum(e, dst, num_segments=N) / jnp.clip(cnt, 1.0)[:, None]
    msg = jax.nn.relu(_bn((a + agg) @ p['am_W'] + p['am_b'], p['am_g'], p['am_beta']))
    h = jnp.concatenate([msg, agg], axis=1)
    h = _gat(h, p['g1_Wl'], p['g1_bl'], p['g1_Wr'], p['g1_br'], p['g1_att'],
             p['g1_bias'], src_pad, off_pad, ts_pad)
    h = jax.nn.relu(_bn(h, p['bn1_g'], p['bn1_b']))
    h = _gat(h, p['g2_Wl'], p['g2_bl'], p['g2_Wr'], p['g2_br'], p['g2_att'],
             p['g2_bias'], src_pad, off_pad, ts_pad)
    h = jax.nn.relu(_bn(h, p['bn2_g'], p['bn2_b']))
    return _mlp_head(h, p)


# speedup vs baseline: 7.4532x; 1.1055x over previous
import functools

import jax
import jax.numpy as jnp
from jax import lax
from jax.experimental import pallas as pl
from jax.experimental.pallas import tpu as pltpu
from jax.experimental.pallas import tpu_sc as plsc

N = 10000
E = 320000
HEADS = 8
HID = 64
HD = HEADS * HID

NTILES = 32
CHUNK = 64
BLK = 2048
SRCCAP = 16384
ETOT = E + N
EPAD = ((ETOT + 4 * CHUNK + 3 * BLK) // BLK + 1) * BLK
OFFPAD = ((N + 1 + 16 + 7) // 8) * 8
TSPAD = 48


def _sload(ref, i):
    base = (i // 8) * 8
    v = ref[pl.ds(base, 16)]
    idx = jnp.full((16, 1), i - base, jnp.int32)
    dn = lax.GatherDimensionNumbers(
        offset_dims=(), collapsed_slice_dims=(0,), start_index_map=(0,))
    return lax.gather(v, idx, dn, (1,),
                      mode=lax.GatherScatterMode.PROMISE_IN_BOUNDS)[0]


def _bn(x, g, b):
    m = jnp.mean(x, axis=0)
    v = jnp.var(x, axis=0)
    return g * (x - m) / jnp.sqrt(v + 1e-5) + b


def _bcast(v, lane):
    idx = jnp.full((16, 1), lane, jnp.int32)
    dn = lax.GatherDimensionNumbers(
        offset_dims=(), collapsed_slice_dims=(0,), start_index_map=(0,))
    return lax.gather(v, idx, dn, (1,),
                      mode=lax.GatherScatterMode.PROMISE_IN_BOUNDS)


def _edge_prep(src, dst):
    loops = jnp.arange(N, dtype=jnp.int32)
    src2 = jnp.concatenate([src.astype(jnp.int32), loops])
    dst2 = jnp.concatenate([dst.astype(jnp.int32), loops])
    key = lax.sort(dst2 * 16384 + src2)
    ss = key & 16383
    off = jnp.searchsorted(key, jnp.arange(N + 1, dtype=jnp.int32) * 16384,
                           side='left').astype(jnp.int32)
    off_pad = jnp.concatenate(
        [off, jnp.full((OFFPAD - (N + 1),), ETOT, jnp.int32)])
    src_pad = jnp.concatenate([ss, jnp.zeros((EPAD - ETOT,), jnp.int32)])
    targets = (jnp.arange(NTILES + 1, dtype=jnp.int32) * ETOT) // NTILES
    ts = jnp.searchsorted(off, targets, side='left').astype(jnp.int32)
    ts_pad = jnp.concatenate([ts, jnp.zeros((TSPAD - (NTILES + 1),), jnp.int32)])
    return src_pad, off_pad, ts_pad


def _gat_body(xl_hbm, xr_hbm, att_hbm, src_hbm, off_hbm, ts_hbm, out_hbm,
              srcbuf, offbuf, tsbuf, attbuf, rowbuf, xibuf, outbuf,
              accbuf, denbuf, sem0, sem1):
    wid = lax.axis_index("s") * 2 + lax.axis_index("c")
    pltpu.sync_copy(off_hbm, offbuf)
    pltpu.sync_copy(ts_hbm, tsbuf)
    pltpu.sync_copy(att_hbm, attbuf)
    n_lo = _sload(tsbuf, wid)
    n_hi = _sload(tsbuf, wid + 1)
    sems = (sem0, sem1)

    def _finalize(i):
        for h in range(HEADS):
            dv = denbuf[pl.ds(h * 16, 16)]
            for kk in range(HID // 16):
                c0 = h * HID + kk * 16
                outbuf[0, pl.ds(c0, 16)] = accbuf[pl.ds(c0, 16)] / dv
        pltpu.sync_copy(outbuf, out_hbm.at[pl.ds(i, 1)])

    def _reset():
        zero = jnp.zeros((16,), jnp.float32)
        for h in range(HEADS):
            denbuf[pl.ds(h * 16, 16)] = zero
        for kk in range(HD // 16):
            accbuf[pl.ds(kk * 16, 16)] = zero

    @pl.when(n_lo < n_hi)
    def _():
        e_lo = _sload(offbuf, n_lo)
        e_hi = _sload(offbuf, n_hi)
        alo = (e_lo // 8) * 8
        nblk = jnp.minimum((e_hi - alo + 4 * CHUNK + BLK - 1) // BLK,
                           SRCCAP // BLK)

        def stage(b, c):
            pltpu.sync_copy(src_hbm.at[pl.ds(alo + b * BLK, BLK)],
                            srcbuf.at[pl.ds(b * BLK, BLK)])
            return c
        lax.fori_loop(0, nblk, stage, 0)

        _reset()
        pltpu.sync_copy(xr_hbm.at[pl.ds(n_lo, 1)], xibuf)
        nchunk = jnp.minimum((e_hi - alo + CHUNK - 1) // CHUNK,
                             SRCCAP // CHUNK - 4)
        npairs = (nchunk + 1) // 2

        def issue(k, slot):
            pltpu.async_copy(
                xl_hbm.at[srcbuf.at[pl.ds(k * CHUNK, CHUNK)]],
                rowbuf.at[slot], sems[slot])

        def wait_slot(k, slot):
            pltpu.make_async_copy(
                xl_hbm.at[srcbuf.at[pl.ds(k * CHUNK, CHUNK)]],
                rowbuf.at[slot], sems[slot]).wait()

        def process(k, slot, carry):
            qbase = alo + k * CHUNK
            ebeg = jnp.maximum(e_lo - qbase, 0)
            ne = jnp.maximum(jnp.minimum(CHUNK, e_hi - qbase), ebeg)

            def edge_body(e, carry):
                cur, bnext = carry
                q = qbase + e
                adv = q == bnext

                @pl.when(adv)
                def _fin():
                    _finalize(cur)
                    _reset()
                    pltpu.sync_copy(xr_hbm.at[pl.ds(cur + 1, 1)], xibuf)

                cur = jnp.where(adv, cur + 1, cur)
                bnext = lax.cond(adv, lambda: _sload(offbuf, cur + 1),
                                 lambda: bnext)
                for h in range(HEADS):
                    sv = jnp.zeros((16,), jnp.float32)
                    xjs = []
                    for kk in range(HID // 16):
                        c0 = h * HID + kk * 16
                        xj = rowbuf[slot, e, pl.ds(c0, 16)]
                        xjs.append(xj)
                        t = xibuf[0, pl.ds(c0, 16)] + xj
                        lk = jnp.maximum(t, 0.2 * t)
                        sv = sv + lk * attbuf[pl.ds(c0, 16)]
                    eb = jnp.exp(jnp.full((16,), jnp.sum(sv)))
                    d0 = pl.ds(h * 16, 16)
                    denbuf[d0] = denbuf[d0] + eb
                    for kk in range(HID // 16):
                        c0 = h * HID + kk * 16
                        accbuf[pl.ds(c0, 16)] = accbuf[pl.ds(c0, 16)] + xjs[kk] * eb
                return (cur, bnext)

            return lax.fori_loop(ebeg, ne, edge_body, carry)

        issue(0, 0)

        def pair_body(p, carry):
            k0 = 2 * p
            k1 = k0 + 1
            wait_slot(k0, 0)
            issue(k1, 1)
            carry = process(k0, 0, carry)
            wait_slot(k1, 1)
            issue(k0 + 2, 0)
            carry = process(k1, 1, carry)
            return carry

        carry0 = (n_lo, _sload(offbuf, n_lo + 1))
        lax.fori_loop(0, npairs, pair_body, carry0)
        wait_slot(2 * npairs, 0)
        _finalize(n_hi - 1)


@jax.jit
def _gat_sc_call(xl, xr, attf, src_pad, off_pad, ts_pad):
    mesh = plsc.VectorSubcoreMesh(core_axis_name="c", subcore_axis_name="s")
    return pl.kernel(
        _gat_body,
        mesh=mesh,
        out_type=jax.ShapeDtypeStruct((N, HD), jnp.float32),
        compiler_params=pltpu.CompilerParams(needs_layout_passes=False),
        scratch_types=[
            pltpu.VMEM((SRCCAP,), jnp.int32),
            pltpu.VMEM((OFFPAD,), jnp.int32),
            pltpu.VMEM((TSPAD,), jnp.int32),
            pltpu.VMEM((HD,), jnp.float32),
            pltpu.VMEM((2, CHUNK, HD), jnp.float32),
            pltpu.VMEM((1, HD), jnp.float32),
            pltpu.VMEM((1, HD), jnp.float32),
            pltpu.VMEM((HD,), jnp.float32),
            pltpu.VMEM((HEADS * 16,), jnp.float32),
            pltpu.SemaphoreType.DMA,
            pltpu.SemaphoreType.DMA,
        ],
    )(xl, xr, attf, src_pad, off_pad, ts_pad)


def _gat(x, Wl, bl, Wr, br, att, bias, src_pad, off_pad, ts_pad):
    xl = x @ Wl + bl
    xr = x @ Wr + br
    out = _gat_sc_call(xl, xr, att.reshape(HD), src_pad, off_pad, ts_pad)
    return out + bias


def _head_kernel(h_ref, p1w_ref, p1g_ref, p1b_ref, p2w_ref, p2g_ref, p2b_ref,
                 p3w_ref, p3b_ref, out_ref):
    h = h_ref[...]
    y = h @ p1w_ref[...]
    m = jnp.mean(y, axis=0)
    v = jnp.mean((y - m[None, :]) ** 2, axis=0)
    y = p1g_ref[...] * (y - m[None, :]) / jnp.sqrt(v[None, :] + 1e-5) + p1b_ref[...]
    y = jnp.maximum(y, 0.0)
    z = y @ p2w_ref[...]
    m2 = jnp.mean(z, axis=0)
    v2 = jnp.mean((z - m2[None, :]) ** 2, axis=0)
    z = p2g_ref[...] * (z - m2[None, :]) / jnp.sqrt(v2[None, :] + 1e-5) + p2b_ref[...]
    z = jnp.maximum(z, 0.0)
    out_ref[...] = z @ p3w_ref[...] + p3b_ref[...]


def _mlp_head(h, p):
    out = pl.pallas_call(
        _head_kernel,
        out_shape=jax.ShapeDtypeStruct((N, 1), jnp.float32),
    )(h, p['p1_W'], p['p1_g'], p['p1_beta'], p['p2_W'], p['p2_g'],
      p['p2_beta'], p['p3_W'], p['p3_b'])
    return out[:, 0]


def kernel(x, edge_index, edge_attr, params):
    p = params
    src, dst = edge_index[0], edge_index[1]
    src_pad, off_pad, ts_pad = _edge_prep(src, dst)
    a = jax.nn.relu(_bn(x @ p['ap_W'] + p['ap_b'], p['ap_g'], p['ap_beta']))
    e = jax.nn.relu(_bn(edge_attr @ p['ep_W'] + p['ep_b'], p['ep_g'], p['ep_beta']))
    cnt = jax.ops.segment_sum(jnp.ones((E,), jnp.float32), dst, num_segments=N)
    agg = jax.ops.segment_sum(e, dst, num_segments=N) / jnp.clip(cnt, 1.0)[:, None]
    msg = jax.nn.relu(_bn((a + agg) @ p['am_W'] + p['am_b'], p['am_g'], p['am_beta']))
    h = jnp.concatenate([msg, agg], axis=1)
    h = _gat(h, p['g1_Wl'], p['g1_bl'], p['g1_Wr'], p['g1_br'], p['g1_att'],
             p['g1_bias'], src_pad, off_pad, ts_pad)
    h = jax.nn.relu(_bn(h, p['bn1_g'], p['bn1_b']))
    h = _gat(h, p['g2_Wl'], p['g2_bl'], p['g2_Wr'], p['g2_br'], p['g2_att'],
             p['g2_bias'], src_pad, off_pad, ts_pad)
    h = jax.nn.relu(_bn(h, p['bn2_g'], p['bn2_b']))
    return _mlp_head(h, p)


# speedup vs baseline: 7.7477x; 1.0395x over previous
import functools

import jax
import jax.numpy as jnp
from jax import lax
from jax.experimental import pallas as pl
from jax.experimental.pallas import tpu as pltpu
from jax.experimental.pallas import tpu_sc as plsc

N = 10000
E = 320000
HEADS = 8
HID = 64
HD = HEADS * HID

NTILES = 32
CHUNK = 64
BLK = 2048
SRCCAP = 16384
ETOT = E + N
EPAD = ((ETOT + 4 * CHUNK + 3 * BLK) // BLK + 1) * BLK
OFFPAD = ((N + 1 + 16 + 7) // 8) * 8
TSPAD = 48


def _sload(ref, i):
    base = (i // 8) * 8
    v = ref[pl.ds(base, 16)]
    idx = jnp.full((16, 1), i - base, jnp.int32)
    dn = lax.GatherDimensionNumbers(
        offset_dims=(), collapsed_slice_dims=(0,), start_index_map=(0,))
    return lax.gather(v, idx, dn, (1,),
                      mode=lax.GatherScatterMode.PROMISE_IN_BOUNDS)[0]


def _bn(x, g, b):
    m = jnp.mean(x, axis=0)
    v = jnp.var(x, axis=0)
    return g * (x - m) / jnp.sqrt(v + 1e-5) + b


def _bcast(v, lane):
    idx = jnp.full((16, 1), lane, jnp.int32)
    dn = lax.GatherDimensionNumbers(
        offset_dims=(), collapsed_slice_dims=(0,), start_index_map=(0,))
    return lax.gather(v, idx, dn, (1,),
                      mode=lax.GatherScatterMode.PROMISE_IN_BOUNDS)


def _edge_prep(src, dst):
    loops = jnp.arange(N, dtype=jnp.int32)
    src2 = jnp.concatenate([src.astype(jnp.int32), loops])
    dst2 = jnp.concatenate([dst.astype(jnp.int32), loops])
    key = lax.sort(dst2 * 16384 + src2)
    ss = key & 16383
    off = jnp.searchsorted(key, jnp.arange(N + 1, dtype=jnp.int32) * 16384,
                           side='left').astype(jnp.int32)
    off_pad = jnp.concatenate(
        [off, jnp.full((OFFPAD - (N + 1),), ETOT, jnp.int32)])
    src_pad = jnp.concatenate([ss, jnp.zeros((EPAD - ETOT,), jnp.int32)])
    targets = (jnp.arange(NTILES + 1, dtype=jnp.int32) * ETOT) // NTILES
    ts = jnp.searchsorted(off, targets, side='left').astype(jnp.int32)
    ts_pad = jnp.concatenate([ts, jnp.zeros((TSPAD - (NTILES + 1),), jnp.int32)])
    cnt = (off[1:] - off[:-1] - 1).astype(jnp.float32)
    return src_pad, off_pad, ts_pad, cnt


AGG_CH = 80
EPS = E // 16
N2 = 10240
NHALF = N2 // 2
NROWS = NHALF // 16


def _agg_body(y_hbm, dst_hbm, z_hbm, out_hbm,
              ybuf, dstbuf, idxbuf, obuf, zbuf, shared, sem):
    c = lax.axis_index("c")
    s = lax.axis_index("s")
    pltpu.sync_copy(z_hbm, zbuf)
    pltpu.sync_copy(zbuf, shared.at[pl.ds(s * NROWS, NROWS)])
    plsc.subcore_barrier()
    lo = c * NHALF
    base = s * EPS

    def chunk(ci, carry):
        st = base + ci * AGG_CH
        pltpu.sync_copy(y_hbm.at[pl.ds(st, AGG_CH)], ybuf)
        pltpu.sync_copy(dst_hbm.at[pl.ds(st, AGG_CH)], dstbuf)
        for o in list(range(0, AGG_CH - 16, 16)) + [AGG_CH - 16]:
            dv = dstbuf[pl.ds(o, 16)]
            keep = (dv >= lo) & (dv < lo + NHALF)
            idxbuf[pl.ds(o, 16)] = jnp.where(keep, dv - lo, NHALF)

        def ebody(e, cc):
            for kk in range(4):
                c0 = kk * 16
                obuf[e, pl.ds(c0, 16)] = jnp.maximum(ybuf[e, pl.ds(c0, 16)], 0.0)
            return cc
        lax.fori_loop(0, AGG_CH, ebody, 0)
        pltpu.sync_copy(obuf, shared.at[idxbuf], add=True)
        return carry
    lax.fori_loop(0, EPS // AGG_CH, chunk, 0)
    plsc.subcore_barrier()
    pltpu.sync_copy(shared.at[pl.ds(s * NROWS, NROWS)],
                    out_hbm.at[pl.ds(c * NHALF + s * NROWS, NROWS)])


@jax.jit
def _agg_sc_call(y, dst, zrows):
    mesh = plsc.VectorSubcoreMesh(core_axis_name="c", subcore_axis_name="s")
    return pl.kernel(
        _agg_body,
        mesh=mesh,
        out_type=jax.ShapeDtypeStruct((N2, 64), jnp.float32),
        compiler_params=pltpu.CompilerParams(needs_layout_passes=False),
        scratch_types=[
            pltpu.VMEM((AGG_CH, 64), jnp.float32),
            pltpu.VMEM((AGG_CH,), jnp.int32),
            pltpu.VMEM((AGG_CH,), jnp.int32),
            pltpu.VMEM((AGG_CH, 64), jnp.float32),
            pltpu.VMEM((NROWS, 64), jnp.float32),
            pltpu.VMEM_SHARED((NHALF + 8, 64), jnp.float32),
            pltpu.SemaphoreType.DMA,
        ],
    )(y, dst, zrows)


def _gat_body(xl_hbm, xr_hbm, att_hbm, src_hbm, off_hbm, ts_hbm, out_hbm,
              srcbuf, offbuf, tsbuf, attbuf, rowbuf, xibuf, outbuf,
              accbuf, denbuf, sem0, sem1):
    wid = lax.axis_index("s") * 2 + lax.axis_index("c")
    pltpu.sync_copy(off_hbm, offbuf)
    pltpu.sync_copy(ts_hbm, tsbuf)
    pltpu.sync_copy(att_hbm, attbuf)
    n_lo = _sload(tsbuf, wid)
    n_hi = _sload(tsbuf, wid + 1)
    sems = (sem0, sem1)

    def _finalize(i):
        for h in range(HEADS):
            dv = denbuf[pl.ds(h * 16, 16)]
            for kk in range(HID // 16):
                c0 = h * HID + kk * 16
                outbuf[0, pl.ds(c0, 16)] = accbuf[pl.ds(c0, 16)] / dv
        pltpu.sync_copy(outbuf, out_hbm.at[pl.ds(i, 1)])

    def _reset():
        zero = jnp.zeros((16,), jnp.float32)
        for h in range(HEADS):
            denbuf[pl.ds(h * 16, 16)] = zero
        for kk in range(HD // 16):
            accbuf[pl.ds(kk * 16, 16)] = zero

    @pl.when(n_lo < n_hi)
    def _():
        e_lo = _sload(offbuf, n_lo)
        e_hi = _sload(offbuf, n_hi)
        alo = (e_lo // 8) * 8
        nblk = jnp.minimum((e_hi - alo + 4 * CHUNK + BLK - 1) // BLK,
                           SRCCAP // BLK)

        def stage(b, c):
            pltpu.sync_copy(src_hbm.at[pl.ds(alo + b * BLK, BLK)],
                            srcbuf.at[pl.ds(b * BLK, BLK)])
            return c
        lax.fori_loop(0, nblk, stage, 0)

        _reset()
        pltpu.sync_copy(xr_hbm.at[pl.ds(n_lo, 1)], xibuf)
        nchunk = jnp.minimum((e_hi - alo + CHUNK - 1) // CHUNK,
                             SRCCAP // CHUNK - 4)
        npairs = (nchunk + 1) // 2

        def issue(k, slot):
            pltpu.async_copy(
                xl_hbm.at[srcbuf.at[pl.ds(k * CHUNK, CHUNK)]],
                rowbuf.at[slot], sems[slot])

        def wait_slot(k, slot):
            pltpu.make_async_copy(
                xl_hbm.at[srcbuf.at[pl.ds(k * CHUNK, CHUNK)]],
                rowbuf.at[slot], sems[slot]).wait()

        def process(k, slot, carry):
            qbase = alo + k * CHUNK
            ebeg = jnp.maximum(e_lo - qbase, 0)
            ne = jnp.maximum(jnp.minimum(CHUNK, e_hi - qbase), ebeg)

            def edge_body(e, carry):
                cur, bnext = carry
                q = qbase + e
                adv = q == bnext

                @pl.when(adv)
                def _fin():
                    _finalize(cur)
                    _reset()
                    pltpu.sync_copy(xr_hbm.at[pl.ds(cur + 1, 1)], xibuf)

                cur = jnp.where(adv, cur + 1, cur)
                bnext = lax.cond(adv, lambda: _sload(offbuf, cur + 1),
                                 lambda: bnext)
                for h in range(HEADS):
                    sv = jnp.zeros((16,), jnp.float32)
                    xjs = []
                    for kk in range(HID // 16):
                        c0 = h * HID + kk * 16
                        xj = rowbuf[slot, e, pl.ds(c0, 16)]
                        xjs.append(xj)
                        t = xibuf[0, pl.ds(c0, 16)] + xj
                        lk = jnp.maximum(t, 0.2 * t)
                        sv = sv + lk * attbuf[pl.ds(c0, 16)]
                    eb = jnp.exp(jnp.full((16,), jnp.sum(sv)))
                    d0 = pl.ds(h * 16, 16)
                    denbuf[d0] = denbuf[d0] + eb
                    for kk in range(HID // 16):
                        c0 = h * HID + kk * 16
                        accbuf[pl.ds(c0, 16)] = accbuf[pl.ds(c0, 16)] + xjs[kk] * eb
                return (cur, bnext)

            return lax.fori_loop(ebeg, ne, edge_body, carry)

        issue(0, 0)

        def pair_body(p, carry):
            k0 = 2 * p
            k1 = k0 + 1
            wait_slot(k0, 0)
            issue(k1, 1)
            carry = process(k0, 0, carry)
            wait_slot(k1, 1)
            issue(k0 + 2, 0)
            carry = process(k1, 1, carry)
            return carry

        carry0 = (n_lo, _sload(offbuf, n_lo + 1))
        lax.fori_loop(0, npairs, pair_body, carry0)
        wait_slot(2 * npairs, 0)
        _finalize(n_hi - 1)


@jax.jit
def _gat_sc_call(xl, xr, attf, src_pad, off_pad, ts_pad):
    mesh = plsc.VectorSubcoreMesh(core_axis_name="c", subcore_axis_name="s")
    return pl.kernel(
        _gat_body,
        mesh=mesh,
        out_type=jax.ShapeDtypeStruct((N, HD), jnp.float32),
        compiler_params=pltpu.CompilerParams(needs_layout_passes=False),
        scratch_types=[
            pltpu.VMEM((SRCCAP,), jnp.int32),
            pltpu.VMEM((OFFPAD,), jnp.int32),
            pltpu.VMEM((TSPAD,), jnp.int32),
            pltpu.VMEM((HD,), jnp.float32),
            pltpu.VMEM((2, CHUNK, HD), jnp.float32),
            pltpu.VMEM((1, HD), jnp.float32),
            pltpu.VMEM((1, HD), jnp.float32),
            pltpu.VMEM((HD,), jnp.float32),
            pltpu.VMEM((HEADS * 16,), jnp.float32),
            pltpu.SemaphoreType.DMA,
            pltpu.SemaphoreType.DMA,
        ],
    )(xl, xr, attf, src_pad, off_pad, ts_pad)


def _gat(x, Wl, bl, Wr, br, att, bias, src_pad, off_pad, ts_pad):
    xl = x @ Wl + bl
    xr = x @ Wr + br
    out = _gat_sc_call(xl, xr, att.reshape(HD), src_pad, off_pad, ts_pad)
    return out + bias


def _head_kernel(h_ref, p1w_ref, p1g_ref, p1b_ref, p2w_ref, p2g_ref, p2b_ref,
                 p3w_ref, p3b_ref, out_ref):
    h = h_ref[...]
    y = h @ p1w_ref[...]
    m = jnp.mean(y, axis=0)
    v = jnp.mean((y - m[None, :]) ** 2, axis=0)
    y = p1g_ref[...] * (y - m[None, :]) / jnp.sqrt(v[None, :] + 1e-5) + p1b_ref[...]
    y = jnp.maximum(y, 0.0)
    z = y @ p2w_ref[...]
    m2 = jnp.mean(z, axis=0)
    v2 = jnp.mean((z - m2[None, :]) ** 2, axis=0)
    z = p2g_ref[...] * (z - m2[None, :]) / jnp.sqrt(v2[None, :] + 1e-5) + p2b_ref[...]
    z = jnp.maximum(z, 0.0)
    out_ref[...] = z @ p3w_ref[...] + p3b_ref[...]


def _mlp_head(h, p):
    out = pl.pallas_call(
        _head_kernel,
        out_shape=jax.ShapeDtypeStruct((N, 1), jnp.float32),
    )(h, p['p1_W'], p['p1_g'], p['p1_beta'], p['p2_W'], p['p2_g'],
      p['p2_beta'], p['p3_W'], p['p3_b'])
    return out[:, 0]


def kernel(x, edge_index, edge_attr, params):
    p = params
    src, dst = edge_index[0], edge_index[1]
    src_pad, off_pad, ts_pad, cnt = _edge_prep(src, dst)
    a = jax.nn.relu(_bn(x @ p['ap_W'] + p['ap_b'], p['ap_g'], p['ap_beta']))
    y = edge_attr @ p['ep_W']
    m = jnp.mean(y, axis=0)
    v = jnp.var(y, axis=0)
    s1 = p['ep_g'] / jnp.sqrt(v + 1e-5)
    y = y * s1[None, :] + (p['ep_beta'] - m * s1)[None, :]
    parts = _agg_sc_call(y, dst.astype(jnp.int32),
                         jnp.zeros((NROWS, 64), jnp.float32))
    agg = parts[:N] / jnp.clip(cnt, 1.0)[:, None]
    msg = jax.nn.relu(_bn((a + agg) @ p['am_W'] + p['am_b'], p['am_g'], p['am_beta']))
    h = jnp.concatenate([msg, agg], axis=1)
    h = _gat(h, p['g1_Wl'], p['g1_bl'], p['g1_Wr'], p['g1_br'], p['g1_att'],
             p['g1_bias'], src_pad, off_pad, ts_pad)
    h = jax.nn.relu(_bn(h, p['bn1_g'], p['bn1_b']))
    h = _gat(h, p['g2_Wl'], p['g2_bl'], p['g2_Wr'], p['g2_br'], p['g2_att'],
             p['g2_bias'], src_pad, off_pad, ts_pad)
    h = jax.nn.relu(_bn(h, p['bn2_g'], p['bn2_b']))
    return _mlp_head(h, p)
